# baseline (speedup 1.0000x reference)
import dataclasses
import functools

import jax
import jax.numpy as jnp
from jax import lax
from jax.experimental import pallas as pl
from jax.experimental.pallas import tpu as pltpu
from jax.experimental.pallas import tpu_sc as plsc

_M = 5000
_W = 256
_MT = 512
_NT = 10
_MP = 5248
_C = 256
_R = 1000
_N = 50000

_NSUB = 16
_HALF = 2504
_TRASH = 2560
_AROWS = 2568
_K = 256
_KI = _K // 16
_CPS = 13
_MB = 160


def _pool_body(ids_ref, s_ref, ws_ref, bs_ref, out_s_ref,
               acc_s_ref, acc_c_ref):
    i = pl.program_id(0)
    nb = pl.num_programs(0)

    @pl.when(i == 0)
    def _():
        acc_s_ref[...] = jnp.zeros_like(acc_s_ref)
        acc_c_ref[...] = jnp.zeros_like(acc_c_ref)

    ids = ids_ref[0]
    mn = jnp.min(ids)
    mx = jnp.max(ids)
    base = (mn // 8) * 8

    sb = s_ref[...].astype(jnp.bfloat16)
    ones = jnp.ones((_R, 8), jnp.bfloat16)

    @pl.when(mx - base < _W)
    def _():
        oh = (base + jax.lax.broadcasted_iota(jnp.int32, (_W, 1), 0)
              == ids).astype(jnp.bfloat16)
        ps = jax.lax.dot_general(oh, sb, (((1,), (0,)), ((), ())),
                                 preferred_element_type=jnp.float32)
        pc = jax.lax.dot_general(oh, ones, (((1,), (0,)), ((), ())),
                                 preferred_element_type=jnp.float32)
        acc_s_ref[pl.ds(base, _W), :] += ps
        acc_c_ref[pl.ds(base, _W), :] += pc

    @pl.when(mx - base >= _W)
    def _():
        for t in range(_NT):
            tb = t * _MT

            @pl.when((mx >= tb) & (mn < tb + _MT))
            def _():
                oh = (tb + jax.lax.broadcasted_iota(jnp.int32, (_MT, 1), 0)
                      == ids).astype(jnp.bfloat16)
                ps = jax.lax.dot_general(oh, sb, (((1,), (0,)), ((), ())),
                                         preferred_element_type=jnp.float32)
                pc = jax.lax.dot_general(oh, ones, (((1,), (0,)), ((), ())),
                                         preferred_element_type=jnp.float32)
                acc_s_ref[pl.ds(tb, _MT), :] += ps
                acc_c_ref[pl.ds(tb, _MT), :] += pc

    @pl.when(i == nb - 1)
    def _():
        denom = jnp.maximum(acc_c_ref[:_M, 0:1], 1.0)
        sm = acc_s_ref[:_M, :] / denom
        out_s_ref[...] = jax.lax.dot_general(
            sm, ws_ref[...], (((1,), (1,)), ((), ())),
            preferred_element_type=jnp.float32) + bs_ref[...]


def _s_path(s, ids, W_s, b_s):
    nb = _N // _R
    ids3 = ids.reshape(nb, 1, _R)
    return pl.pallas_call(
        _pool_body,
        grid=(nb,),
        in_specs=[
            pl.BlockSpec((1, 1, _R), lambda i: (i, 0, 0)),
            pl.BlockSpec((_R, _C), lambda i: (i, 0)),
            pl.BlockSpec((_C, _C), lambda i: (0, 0)),
            pl.BlockSpec((1, _C), lambda i: (0, 0)),
        ],
        out_specs=pl.BlockSpec((_M, _C), lambda i: (0, 0)),
        out_shape=jax.ShapeDtypeStruct((_M, _C), jnp.float32),
        scratch_shapes=[
            pltpu.VMEM((_MP, _C), jnp.float32),
            pltpu.VMEM((_MP, 8), jnp.float32),
        ],
        compiler_params=pltpu.CompilerParams(
            dimension_semantics=("arbitrary",)),
    )(ids3, s, W_s, b_s.reshape(1, _C))


def _iota16():
    return lax.iota(jnp.int32, 16)


def _full16(x):
    return jnp.broadcast_to(x, (16,)).astype(jnp.int32)


def _v_body(v_hbm, ids_hbm, wvt_hbm, bv_hbm, out_hbm,
            acc_sh, cnt_sh, vbuf, ids_raw, idx2, vones,
            vacc, vcnt, vout, wvt_v, bv_v):
    cid = lax.axis_index("c")
    sid = lax.axis_index("s")
    mbase = cid * _HALF
    zero16 = jnp.zeros((16,), jnp.float32)
    one16 = jnp.ones((16,), jnp.float32)

    @pl.loop(0, _MB)
    def _(r):
        plsc.store_scatter(vcnt, [_full16(r), _iota16()], zero16)
        for d in range(3):
            plsc.store_scatter(vout, [_full16(r), _iota16(), _full16(d)],
                               zero16)

    @pl.loop(0, _K)
    def _(r):
        plsc.store_scatter(vones, [_full16(r), _iota16()], one16)

    pltpu.sync_copy(vout, acc_sh.at[pl.ds(sid * _MB, _MB)])
    pltpu.sync_copy(vcnt, cnt_sh.at[pl.ds(sid * _MB, _MB)])

    @pl.when(sid == 0)
    def _():
        pltpu.sync_copy(vout.at[pl.ds(0, 8)],
                        acc_sh.at[pl.ds(_TRASH, 8)])
        pltpu.sync_copy(vcnt.at[pl.ds(0, 8)],
                        cnt_sh.at[pl.ds(_TRASH, 8)])

    pltpu.sync_copy(wvt_hbm, wvt_v)
    pltpu.sync_copy(bv_hbm, bv_v)
    plsc.subcore_barrier()

    def do_chunk(r0, nvec):
        pltpu.sync_copy(ids_hbm.at[pl.ds(r0, 16 * nvec)],
                        ids_raw.at[pl.ds(0, 16 * nvec)])
        pltpu.sync_copy(v_hbm.at[pl.ds(r0, 16 * nvec)],
                        vbuf.at[pl.ds(0, 16 * nvec)])
        for i in range(_KI):
            if i < nvec:
                vec = ids_raw[pl.ds(16 * i, 16)]
                loc = vec - _full16(mbase)
                ok = (loc >= 0) & (loc < _HALF)
                idx = jnp.where(ok, loc, _TRASH)
            else:
                idx = _full16(_TRASH)
            idx2[i // 8, pl.ds(16 * (i % 8), 16)] = idx
        for j in range(_K // 128):
            pltpu.sync_copy(vbuf.at[pl.ds(128 * j, 128)],
                            acc_sh.at[idx2.at[j]], add=True)
            pltpu.sync_copy(vones.at[pl.ds(128 * j, 128)],
                            cnt_sh.at[idx2.at[j]], add=True)

    @pl.loop(0, _CPS)
    def _(u):
        r0 = sid * (_CPS * _K) + u * _K

        @pl.when(r0 + _K <= _N)
        def _():
            do_chunk(r0, _KI)

        @pl.when(r0 == (_N // _K) * _K)
        def _():
            do_chunk(r0, (_N - (_N // _K) * _K) // 16)

    plsc.subcore_barrier()

    pltpu.sync_copy(acc_sh.at[pl.ds(sid * _MB, _MB)], vacc)
    pltpu.sync_copy(cnt_sh.at[pl.ds(sid * _MB, _MB)], vcnt)
    bv = bv_v[:]

    @pl.loop(0, _MB // 16)
    def _(mb):
        m0 = mb * 16
        midx = _full16(m0) + _iota16()
        cnt = plsc.load_gather(vcnt, [midx, _full16(0)])
        inv = 1.0 / jnp.maximum(cnt, 1.0)
        for d in range(3):
            vmk = [plsc.load_gather(vacc, [midx, _full16(k), _full16(d)]) * inv
                   for k in range(16)]
            for j in range(16):
                wrow = wvt_v[j, :]
                o = jnp.broadcast_to(bv[j], (16,))
                for k in range(16):
                    o = o + jnp.broadcast_to(wrow[k], (16,)) * vmk[k]
                plsc.store_scatter(vout, [midx, _full16(j), _full16(d)], o)

    gbase = mbase + sid * _MB

    @pl.when(sid < _NSUB - 1)
    def _():
        pltpu.sync_copy(vout, out_hbm.at[pl.ds(gbase, _MB)])

    @pl.when((sid == _NSUB - 1) & (cid == 0))
    def _():
        pltpu.sync_copy(vout.at[pl.ds(0, _HALF - 15 * _MB)],
                        out_hbm.at[pl.ds(gbase, _HALF - 15 * _MB)])

    @pl.when((sid == _NSUB - 1) & (cid == 1))
    def _():
        pltpu.sync_copy(vout.at[pl.ds(0, _M - _HALF - 15 * _MB)],
                        out_hbm.at[pl.ds(gbase, _M - _HALF - 15 * _MB)])


def _v_path(v, ids, W_v, b_v):
    mesh = plsc.VectorSubcoreMesh(core_axis_name="c", subcore_axis_name="s")
    cp = pltpu.CompilerParams()
    if "needs_layout_passes" in pltpu.CompilerParams.__dataclass_fields__:
        cp = dataclasses.replace(cp, needs_layout_passes=False)
    if "use_tc_tiling_on_sc" in pltpu.CompilerParams.__dataclass_fields__:
        cp = dataclasses.replace(cp, use_tc_tiling_on_sc=False)
    kern = functools.partial(
        pl.kernel,
        mesh=mesh,
        compiler_params=cp,
        out_type=jax.ShapeDtypeStruct((_M, 16, 3), jnp.float32),
        scratch_types=[
            pltpu.VMEM_SHARED((_AROWS, 16, 3), jnp.float32),
            pltpu.VMEM_SHARED((_AROWS, 16), jnp.float32),
            pltpu.VMEM((_K, 16, 3), jnp.float32),
            pltpu.VMEM((_K,), jnp.int32),
            pltpu.VMEM((_K // 128, 128), jnp.int32),
            pltpu.VMEM((_K, 16), jnp.float32),
            pltpu.VMEM((_MB, 16, 3), jnp.float32),
            pltpu.VMEM((_MB, 16), jnp.float32),
            pltpu.VMEM((_MB, 16, 3), jnp.float32),
            pltpu.VMEM((16, 16), jnp.float32),
            pltpu.VMEM((16,), jnp.float32),
        ],
    )(_v_body)
    return kern(v, ids, W_v, b_v)


def kernel(s, v, motif_batch, W_s, b_s, W_v, b_v):
    ids = motif_batch.astype(jnp.int32)
    out_s = _s_path(s, ids, W_s, b_s)
    out_v = _v_path(v, ids, W_v, b_v)
    return out_s, out_v


# speedup vs baseline: 12.8916x; 12.8916x over previous
import dataclasses
import functools

import jax
import jax.numpy as jnp
from jax import lax
from jax.experimental import pallas as pl
from jax.experimental.pallas import tpu as pltpu
from jax.experimental.pallas import tpu_sc as plsc

_M = 5000
_W = 256
_MT = 512
_NT = 10
_MP = 5248
_C = 256
_R = 1000
_N = 50000

_NSUB = 16
_HALF = 2504
_TRASH = 2560
_AROWS = 2568
_K = 128
_KI = _K // 16
_CPS = 25
_MB = 160
_VR = 128


def _pool_body(ids_ref, s_ref, ws_ref, bs_ref, out_s_ref,
               acc_s_ref, acc_c_ref):
    i = pl.program_id(0)
    nb = pl.num_programs(0)

    @pl.when(i == 0)
    def _():
        acc_s_ref[...] = jnp.zeros_like(acc_s_ref)
        acc_c_ref[...] = jnp.zeros_like(acc_c_ref)

    ids = ids_ref[0]
    mn = jnp.min(ids)
    mx = jnp.max(ids)
    base = (mn // 8) * 8

    sb = s_ref[...].astype(jnp.bfloat16)
    ones = jnp.ones((_R, 8), jnp.bfloat16)

    @pl.when(mx - base < _W)
    def _():
        oh = (base + jax.lax.broadcasted_iota(jnp.int32, (_W, 1), 0)
              == ids).astype(jnp.bfloat16)
        ps = jax.lax.dot_general(oh, sb, (((1,), (0,)), ((), ())),
                                 preferred_element_type=jnp.float32)
        pc = jax.lax.dot_general(oh, ones, (((1,), (0,)), ((), ())),
                                 preferred_element_type=jnp.float32)
        acc_s_ref[pl.ds(base, _W), :] += ps
        acc_c_ref[pl.ds(base, _W), :] += pc

    @pl.when(mx - base >= _W)
    def _():
        for t in range(_NT):
            tb = t * _MT

            @pl.when((mx >= tb) & (mn < tb + _MT))
            def _():
                oh = (tb + jax.lax.broadcasted_iota(jnp.int32, (_MT, 1), 0)
                      == ids).astype(jnp.bfloat16)
                ps = jax.lax.dot_general(oh, sb, (((1,), (0,)), ((), ())),
                                         preferred_element_type=jnp.float32)
                pc = jax.lax.dot_general(oh, ones, (((1,), (0,)), ((), ())),
                                         preferred_element_type=jnp.float32)
                acc_s_ref[pl.ds(tb, _MT), :] += ps
                acc_c_ref[pl.ds(tb, _MT), :] += pc

    @pl.when(i == nb - 1)
    def _():
        denom = jnp.maximum(acc_c_ref[:_M, 0:1], 1.0)
        sm = acc_s_ref[:_M, :] / denom
        out_s_ref[...] = jax.lax.dot_general(
            sm, ws_ref[...], (((1,), (1,)), ((), ())),
            preferred_element_type=jnp.float32) + bs_ref[...]


def _s_path(s, ids, W_s, b_s):
    nb = _N // _R
    ids3 = ids.reshape(nb, 1, _R)
    return pl.pallas_call(
        _pool_body,
        grid=(nb,),
        in_specs=[
            pl.BlockSpec((1, 1, _R), lambda i: (i, 0, 0)),
            pl.BlockSpec((_R, _C), lambda i: (i, 0)),
            pl.BlockSpec((_C, _C), lambda i: (0, 0)),
            pl.BlockSpec((1, _C), lambda i: (0, 0)),
        ],
        out_specs=pl.BlockSpec((_M, _C), lambda i: (0, 0)),
        out_shape=jax.ShapeDtypeStruct((_M, _C), jnp.float32),
        scratch_shapes=[
            pltpu.VMEM((_MP, _C), jnp.float32),
            pltpu.VMEM((_MP, 8), jnp.float32),
        ],
        compiler_params=pltpu.CompilerParams(
            dimension_semantics=("arbitrary",)),
    )(ids3, s, W_s, b_s.reshape(1, _C))


def _relayout_body(vt_ref, out_ref):
    e = (jax.lax.broadcasted_iota(jnp.int32, (48, _VR), 0)
         == jax.lax.broadcasted_iota(jnp.int32, (48, _VR), 1))
    out_ref[...] = jax.lax.dot_general(
        vt_ref[...], e.astype(jnp.float32), (((0,), (0,)), ((), ())),
        preferred_element_type=jnp.float32)


def _v_relayout(v):
    npad = 51200
    vp = jnp.pad(v, ((0, npad - _N), (0, 0), (0, 0)))
    vt = vp.transpose(2, 1, 0).reshape(48, npad)
    nb = 25
    rb = npad // nb
    out = pl.pallas_call(
        _relayout_body,
        grid=(nb,),
        in_specs=[pl.BlockSpec((48, rb), lambda i: (0, i))],
        out_specs=pl.BlockSpec((rb, _VR), lambda i: (i, 0)),
        out_shape=jax.ShapeDtypeStruct((npad, _VR), jnp.float32),
    )(vt)
    return out.reshape(npad * _VR)


def _iota16():
    return lax.iota(jnp.int32, 16)


def _full16(x):
    return jnp.broadcast_to(x, (16,)).astype(jnp.int32)


def _v_body(v_hbm, ids_hbm, wvt_hbm, bv_hbm, out_hbm,
            acc_sh, cnt_sh, vraw, vbuf, ids_raw, idx2, vones,
            vacc, vcnt, vout1, wvt_v, bv_v):
    cid = lax.axis_index("c")
    sid = lax.axis_index("s")
    mbase = cid * _HALF
    zero16 = jnp.zeros((16,), jnp.float32)
    one16 = jnp.ones((16,), jnp.float32)

    @pl.loop(0, _MB)
    def _(r):
        plsc.store_scatter(vcnt, [_full16(r), _iota16()], zero16)
        for c in range(3):
            plsc.store_scatter(vacc, [_full16(r), 16 * c + _iota16()],
                               zero16)

    @pl.loop(0, _K)
    def _(r):
        plsc.store_scatter(vones, [_full16(r), _iota16()], one16)

    pltpu.sync_copy(vacc, acc_sh.at[pl.ds(sid * _MB, _MB)])
    pltpu.sync_copy(vcnt, cnt_sh.at[pl.ds(sid * _MB, _MB)])

    @pl.when(sid == 0)
    def _():
        pltpu.sync_copy(vacc.at[pl.ds(0, 8)],
                        acc_sh.at[pl.ds(_TRASH, 8)])
        pltpu.sync_copy(vcnt.at[pl.ds(0, 8)],
                        cnt_sh.at[pl.ds(_TRASH, 8)])

    pltpu.sync_copy(wvt_hbm, wvt_v)
    pltpu.sync_copy(bv_hbm, bv_v)
    plsc.subcore_barrier()

    def do_chunk(r0, nvec):
        pltpu.sync_copy(ids_hbm.at[pl.ds(r0, 16 * nvec)],
                        ids_raw.at[pl.ds(0, 16 * nvec)])
        pltpu.sync_copy(v_hbm.at[pl.ds(_VR * r0, _VR * 16 * nvec)],
                        vraw.at[pl.ds(0, _VR * 16 * nvec)])
        for i in range(_KI):
            if i < nvec:
                vec = ids_raw[pl.ds(16 * i, 16)]
                loc = vec - _full16(mbase)
                ok = (loc >= 0) & (loc < _HALF)
                idx = jnp.where(ok, loc, _TRASH)
            else:
                idx = _full16(_TRASH)
            idx2[i // 8, pl.ds(16 * (i % 8), 16)] = idx

        @pl.loop(0, 16 * nvec)
        def _(r):
            for c in range(3):
                vbuf[r, pl.ds(16 * c, 16)] = vraw[pl.ds(_VR * r + 16 * c,
                                                        16)]

        for j in range(_K // 128):
            pltpu.sync_copy(vbuf.at[pl.ds(128 * j, 128)],
                            acc_sh.at[idx2.at[j]], add=True)
            pltpu.sync_copy(vones.at[pl.ds(128 * j, 128)],
                            cnt_sh.at[idx2.at[j]], add=True)

    @pl.loop(0, _CPS)
    def _(u):
        r0 = sid * (_CPS * _K) + u * _K

        @pl.when(r0 + _K <= _N)
        def _():
            do_chunk(r0, _KI)

        @pl.when(r0 == (_N // _K) * _K)
        def _():
            do_chunk(r0, (_N - (_N // _K) * _K) // 16)

    plsc.subcore_barrier()

    pltpu.sync_copy(acc_sh.at[pl.ds(sid * _MB, _MB)], vacc)
    pltpu.sync_copy(cnt_sh.at[pl.ds(sid * _MB, _MB)], vcnt)
    bv = bv_v[:]
    i48 = 48 * _iota16()

    @pl.loop(0, _MB // 16)
    def _(mb):
        m0 = mb * 16
        midx = _full16(m0) + _iota16()
        cnt = plsc.load_gather(vcnt, [midx, _full16(0)])
        inv = 1.0 / jnp.maximum(cnt, 1.0)
        for d in range(3):
            vmk = [plsc.load_gather(vacc, [midx, _full16(16 * d + k)]) * inv
                   for k in range(16)]
            for j in range(16):
                wrow = wvt_v[j, :]
                o = jnp.broadcast_to(bv[j], (16,))
                for k in range(16):
                    o = o + jnp.broadcast_to(wrow[k], (16,)) * vmk[k]
                plsc.store_scatter(
                    vout1, [i48 + _full16(48 * m0 + 3 * j + d)], o)

    gbase = mbase + sid * _MB

    @pl.when(sid < _NSUB - 1)
    def _():
        pltpu.sync_copy(vout1, out_hbm.at[pl.ds(48 * gbase, 48 * _MB)])

    @pl.when((sid == _NSUB - 1) & (cid == 0))
    def _():
        pltpu.sync_copy(vout1.at[pl.ds(0, 48 * (_HALF - 15 * _MB))],
                        out_hbm.at[pl.ds(48 * gbase,
                                         48 * (_HALF - 15 * _MB))])

    @pl.when((sid == _NSUB - 1) & (cid == 1))
    def _():
        pltpu.sync_copy(vout1.at[pl.ds(0, 48 * (_M - _HALF - 15 * _MB))],
                        out_hbm.at[pl.ds(48 * gbase,
                                         48 * (_M - _HALF - 15 * _MB))])


def _v_path(v, ids, W_v, b_v):
    mesh = plsc.VectorSubcoreMesh(core_axis_name="c", subcore_axis_name="s")
    cp = pltpu.CompilerParams()
    if "needs_layout_passes" in pltpu.CompilerParams.__dataclass_fields__:
        cp = dataclasses.replace(cp, needs_layout_passes=False)
    if "use_tc_tiling_on_sc" in pltpu.CompilerParams.__dataclass_fields__:
        cp = dataclasses.replace(cp, use_tc_tiling_on_sc=False)
    kern = functools.partial(
        pl.kernel,
        mesh=mesh,
        compiler_params=cp,
        out_type=jax.ShapeDtypeStruct((_M * 48,), jnp.float32),
        scratch_types=[
            pltpu.VMEM_SHARED((_AROWS, 48), jnp.float32),
            pltpu.VMEM_SHARED((_AROWS, 16), jnp.float32),
            pltpu.VMEM((_K * _VR,), jnp.float32),
            pltpu.VMEM((_K, 48), jnp.float32),
            pltpu.VMEM((_K,), jnp.int32),
            pltpu.VMEM((_K // 128, 128), jnp.int32),
            pltpu.VMEM((_K, 16), jnp.float32),
            pltpu.VMEM((_MB, 48), jnp.float32),
            pltpu.VMEM((_MB, 16), jnp.float32),
            pltpu.VMEM((_MB * 48,), jnp.float32),
            pltpu.VMEM((16, 16), jnp.float32),
            pltpu.VMEM((16,), jnp.float32),
        ],
    )(_v_body)
    out = kern(v, ids, W_v, b_v)
    return out.reshape(_M, 16, 3)


def kernel(s, v, motif_batch, W_s, b_s, W_v, b_v):
    ids = motif_batch.astype(jnp.int32)
    v128 = _v_relayout(v)
    out_s = _s_path(s, ids, W_s, b_s)
    out_v = _v_path(v128, ids, W_v, b_v)
    return out_s, out_v


# speedup vs baseline: 17.2097x; 1.3350x over previous
import dataclasses
import functools

import jax
import jax.numpy as jnp
from jax import lax
from jax.experimental import pallas as pl
from jax.experimental.pallas import tpu as pltpu
from jax.experimental.pallas import tpu_sc as plsc

_M = 5000
_W = 256
_MT = 512
_NT = 10
_MP = 5248
_C = 256
_R = 1000
_N = 50000

_NSUB = 16
_HALF = 2504
_TRASH = 2560
_AROWS = 2568
_K = 128
_KI = _K // 16
_CPS = 25
_MB = 160
_VR = 128


def _pool_body(ids_ref, s_ref, ws_ref, bs_ref, out_s_ref,
               acc_s_ref, acc_c_ref):
    i = pl.program_id(0)
    nb = pl.num_programs(0)

    @pl.when(i == 0)
    def _():
        acc_s_ref[...] = jnp.zeros_like(acc_s_ref)
        acc_c_ref[...] = jnp.zeros_like(acc_c_ref)

    ids = ids_ref[0]
    mn = jnp.min(ids)
    mx = jnp.max(ids)
    base = (mn // 8) * 8

    sb = s_ref[...].astype(jnp.bfloat16)
    ones = jnp.ones((_R, 8), jnp.bfloat16)

    @pl.when(mx - base < _W)
    def _():
        oh = (base + jax.lax.broadcasted_iota(jnp.int32, (_W, 1), 0)
              == ids).astype(jnp.bfloat16)
        ps = jax.lax.dot_general(oh, sb, (((1,), (0,)), ((), ())),
                                 preferred_element_type=jnp.float32)
        pc = jax.lax.dot_general(oh, ones, (((1,), (0,)), ((), ())),
                                 preferred_element_type=jnp.float32)
        acc_s_ref[pl.ds(base, _W), :] += ps
        acc_c_ref[pl.ds(base, _W), :] += pc

    @pl.when(mx - base >= _W)
    def _():
        for t in range(_NT):
            tb = t * _MT

            @pl.when((mx >= tb) & (mn < tb + _MT))
            def _():
                oh = (tb + jax.lax.broadcasted_iota(jnp.int32, (_MT, 1), 0)
                      == ids).astype(jnp.bfloat16)
                ps = jax.lax.dot_general(oh, sb, (((1,), (0,)), ((), ())),
                                         preferred_element_type=jnp.float32)
                pc = jax.lax.dot_general(oh, ones, (((1,), (0,)), ((), ())),
                                         preferred_element_type=jnp.float32)
                acc_s_ref[pl.ds(tb, _MT), :] += ps
                acc_c_ref[pl.ds(tb, _MT), :] += pc

    @pl.when(i == nb - 1)
    def _():
        denom = jnp.maximum(acc_c_ref[:_M, 0:1], 1.0)
        sm = acc_s_ref[:_M, :] / denom
        out_s_ref[...] = jax.lax.dot_general(
            sm, ws_ref[...], (((1,), (1,)), ((), ())),
            preferred_element_type=jnp.float32) + bs_ref[...]


def _s_path(s, ids, W_s, b_s):
    nb = _N // _R
    ids3 = ids.reshape(nb, 1, _R)
    return pl.pallas_call(
        _pool_body,
        grid=(nb,),
        in_specs=[
            pl.BlockSpec((1, 1, _R), lambda i: (i, 0, 0)),
            pl.BlockSpec((_R, _C), lambda i: (i, 0)),
            pl.BlockSpec((_C, _C), lambda i: (0, 0)),
            pl.BlockSpec((1, _C), lambda i: (0, 0)),
        ],
        out_specs=pl.BlockSpec((_M, _C), lambda i: (0, 0)),
        out_shape=jax.ShapeDtypeStruct((_M, _C), jnp.float32),
        scratch_shapes=[
            pltpu.VMEM((_MP, _C), jnp.float32),
            pltpu.VMEM((_MP, 8), jnp.float32),
        ],
        compiler_params=pltpu.CompilerParams(
            dimension_semantics=("arbitrary",)),
    )(ids3, s, W_s, b_s.reshape(1, _C))


def _relayout_body(vt_ref, out_ref):
    e = (jax.lax.broadcasted_iota(jnp.int32, (48, _VR), 0)
         == jax.lax.broadcasted_iota(jnp.int32, (48, _VR), 1))
    out_ref[...] = jax.lax.dot_general(
        vt_ref[...], e.astype(jnp.float32), (((0,), (0,)), ((), ())),
        preferred_element_type=jnp.float32)


def _v_relayout(v):
    npad = 51200
    vp = jnp.pad(v, ((0, npad - _N), (0, 0), (0, 0)))
    vt = vp.transpose(2, 1, 0).reshape(48, npad)
    nb = 25
    rb = npad // nb
    out = pl.pallas_call(
        _relayout_body,
        grid=(nb,),
        in_specs=[pl.BlockSpec((48, rb), lambda i: (0, i))],
        out_specs=pl.BlockSpec((rb, _VR), lambda i: (i, 0)),
        out_shape=jax.ShapeDtypeStruct((npad, _VR), jnp.float32),
    )(vt)
    return out.reshape(npad * _VR)


def _iota16():
    return lax.iota(jnp.int32, 16)


def _full16(x):
    return jnp.broadcast_to(x, (16,)).astype(jnp.int32)


def _v_body(v_hbm, ids_hbm, wvt_hbm, bv_hbm, out_hbm,
            acc_sh, cnt_sh, vraw, vbuf, ids_raw, idx2, vones,
            vacc, vcnt, vout1, wvt_v, bv_v):
    cid = lax.axis_index("c")
    sid = lax.axis_index("s")
    mbase = cid * _HALF
    zero16 = jnp.zeros((16,), jnp.float32)
    one16 = jnp.ones((16,), jnp.float32)

    @pl.loop(0, _MB)
    def _(r):
        plsc.store_scatter(vcnt, [_full16(r), _iota16()], zero16)
        for c in range(3):
            plsc.store_scatter(vacc, [_full16(r), 16 * c + _iota16()],
                               zero16)

    @pl.loop(0, _K)
    def _(r):
        plsc.store_scatter(vones, [_full16(r), _iota16()], one16)

    pltpu.sync_copy(vacc, acc_sh.at[pl.ds(sid * _MB, _MB)])
    pltpu.sync_copy(vcnt, cnt_sh.at[pl.ds(sid * _MB, _MB)])

    @pl.when(sid == 0)
    def _():
        pltpu.sync_copy(vacc.at[pl.ds(0, 8)],
                        acc_sh.at[pl.ds(_TRASH, 8)])
        pltpu.sync_copy(vcnt.at[pl.ds(0, 8)],
                        cnt_sh.at[pl.ds(_TRASH, 8)])

    pltpu.sync_copy(wvt_hbm, wvt_v)
    pltpu.sync_copy(bv_hbm, bv_v)
    plsc.subcore_barrier()

    def do_chunk(r0, nvec):
        pltpu.sync_copy(ids_hbm.at[pl.ds(r0, 16 * nvec)],
                        ids_raw.at[pl.ds(0, 16 * nvec)])
        pltpu.sync_copy(v_hbm.at[pl.ds(_VR * r0, _VR * 16 * nvec)],
                        vraw.at[pl.ds(0, _VR * 16 * nvec)])
        for i in range(_KI):
            if i < nvec:
                vec = ids_raw[pl.ds(16 * i, 16)]
                loc = vec - _full16(mbase)
                ok = (loc >= 0) & (loc < _HALF)
                idx = jnp.where(ok, loc, _TRASH)
            else:
                idx = _full16(_TRASH)
            idx2[i // 8, pl.ds(16 * (i % 8), 16)] = idx

        @pl.loop(0, 16 * nvec)
        def _(r):
            for c in range(3):
                vbuf[r, pl.ds(16 * c, 16)] = vraw[pl.ds(_VR * r + 16 * c,
                                                        16)]

        for j in range(_K // 128):
            pltpu.sync_copy(vbuf.at[pl.ds(128 * j, 128)],
                            acc_sh.at[idx2.at[j]], add=True)
            pltpu.sync_copy(vones.at[pl.ds(128 * j, 128)],
                            cnt_sh.at[idx2.at[j]], add=True)

    @pl.loop(0, _CPS)
    def _(u):
        r0 = sid * (_CPS * _K) + u * _K

        @pl.when(r0 + _K <= _N)
        def _():
            do_chunk(r0, _KI)

        @pl.when(r0 == (_N // _K) * _K)
        def _():
            do_chunk(r0, (_N - (_N // _K) * _K) // 16)

    plsc.subcore_barrier()

    pltpu.sync_copy(acc_sh.at[pl.ds(sid * _MB, _MB)], vacc)
    pltpu.sync_copy(cnt_sh.at[pl.ds(sid * _MB, _MB)], vcnt)
    bv = bv_v[:]

    @pl.loop(0, _MB // 16)
    def _(mb):
        m0 = mb * 16
        midx = _full16(m0) + _iota16()
        cnt = plsc.load_gather(vcnt, [midx, _full16(0)])
        inv = 1.0 / jnp.maximum(cnt, 1.0)
        for d in range(3):
            vmk = [plsc.load_gather(vacc, [midx, _full16(16 * d + k)]) * inv
                   for k in range(16)]
            for j in range(16):
                wrow = wvt_v[j, :]
                o = jnp.broadcast_to(bv[j], (16,))
                for k in range(16):
                    o = o + jnp.broadcast_to(wrow[k], (16,)) * vmk[k]
                plsc.store_scatter(
                    vout1, [_iota16() + _full16((16 * d + j) * _MB + m0)], o)

    gbase = mbase + sid * _MB

    @pl.when(sid < _NSUB - 1)
    def _():
        for p in range(48):
            pltpu.sync_copy(vout1.at[pl.ds(_MB * p, _MB)],
                            out_hbm.at[pl.ds(_M * p + gbase, _MB)])

    @pl.when((sid == _NSUB - 1) & (cid == 0))
    def _():
        for p in range(48):
            pltpu.sync_copy(vout1.at[pl.ds(_MB * p, _HALF - 15 * _MB)],
                            out_hbm.at[pl.ds(_M * p + gbase,
                                             _HALF - 15 * _MB)])

    @pl.when((sid == _NSUB - 1) & (cid == 1))
    def _():
        for p in range(48):
            pltpu.sync_copy(vout1.at[pl.ds(_MB * p, _M - _HALF - 15 * _MB)],
                            out_hbm.at[pl.ds(_M * p + gbase,
                                             _M - _HALF - 15 * _MB)])


def _v_path(v, ids, W_v, b_v):
    mesh = plsc.VectorSubcoreMesh(core_axis_name="c", subcore_axis_name="s")
    cp = pltpu.CompilerParams()
    if "needs_layout_passes" in pltpu.CompilerParams.__dataclass_fields__:
        cp = dataclasses.replace(cp, needs_layout_passes=False)
    if "use_tc_tiling_on_sc" in pltpu.CompilerParams.__dataclass_fields__:
        cp = dataclasses.replace(cp, use_tc_tiling_on_sc=False)
    kern = functools.partial(
        pl.kernel,
        mesh=mesh,
        compiler_params=cp,
        out_type=jax.ShapeDtypeStruct((_M * 48,), jnp.float32),
        scratch_types=[
            pltpu.VMEM_SHARED((_AROWS, 48), jnp.float32),
            pltpu.VMEM_SHARED((_AROWS, 16), jnp.float32),
            pltpu.VMEM((_K * _VR,), jnp.float32),
            pltpu.VMEM((_K, 48), jnp.float32),
            pltpu.VMEM((_K,), jnp.int32),
            pltpu.VMEM((_K // 128, 128), jnp.int32),
            pltpu.VMEM((_K, 16), jnp.float32),
            pltpu.VMEM((_MB, 48), jnp.float32),
            pltpu.VMEM((_MB, 16), jnp.float32),
            pltpu.VMEM((_MB * 48,), jnp.float32),
            pltpu.VMEM((16, 16), jnp.float32),
            pltpu.VMEM((16,), jnp.float32),
        ],
    )(_v_body)
    out = kern(v, ids, W_v, b_v)
    return out.reshape(3, 16, _M).transpose(2, 1, 0)


def kernel(s, v, motif_batch, W_s, b_s, W_v, b_v):
    ids = motif_batch.astype(jnp.int32)
    v128 = _v_relayout(v)
    out_s = _s_path(s, ids, W_s, b_s)
    out_v = _v_path(v128, ids, W_v, b_v)
    return out_s, out_v
